# baseline (speedup 1.0000x reference)
import jax
import jax.numpy as jnp
from jax import lax
from jax.experimental import pallas as pl
from jax.experimental.pallas import tpu as pltpu

_ACTIONS = 8
_BLOCK_N = 16384


def _mlp_t_kernel(xt_ref, w1_ref, b1t_ref, w2s_ref, b2t_ref, o_ref):
    ht = lax.dot_general(w1_ref[...], xt_ref[...],
                         (((0,), (0,)), ((), ())),
                         preferred_element_type=jnp.float32)
    ht = jnp.tanh(ht + b1t_ref[...])
    qt = lax.dot_general(w2s_ref[...], ht,
                         (((0,), (0,)), ((), ())),
                         preferred_element_type=jnp.float32)
    o_ref[...] = (qt + b2t_ref[...]).astype(o_ref.dtype)


def kernel(x, w1, b1, w2p, b2p):
    B, F = x.shape
    H = w1.shape[1]

    xt = x.T
    b1t = b1.T
    w2s = w2p[:, :_ACTIONS]
    b2t = b2p[:, :_ACTIONS].T

    bn = min(_BLOCK_N, B)
    nb = pl.cdiv(B, bn)
    bp = nb * bn
    if bp != B:
        xt = jnp.pad(xt, ((0, 0), (0, bp - B)))

    flops = 2 * bp * (F * H + H * _ACTIONS)
    bytes_accessed = 4 * (bp * F + F * H + H + H * _ACTIONS + _ACTIONS
                          + bp * _ACTIONS)
    qt = pl.pallas_call(
        _mlp_t_kernel,
        out_shape=jax.ShapeDtypeStruct((_ACTIONS, bp), jnp.float32),
        grid=(nb,),
        in_specs=[
            pl.BlockSpec((F, bn), lambda i: (0, i)),
            pl.BlockSpec((F, H), lambda i: (0, 0)),
            pl.BlockSpec((H, 1), lambda i: (0, 0)),
            pl.BlockSpec((H, _ACTIONS), lambda i: (0, 0)),
            pl.BlockSpec((_ACTIONS, 1), lambda i: (0, 0)),
        ],
        out_specs=pl.BlockSpec((_ACTIONS, bn), lambda i: (0, i)),
        compiler_params=pltpu.CompilerParams(
            dimension_semantics=("parallel",)),
        cost_estimate=pl.CostEstimate(flops=flops,
                                      transcendentals=bp * H,
                                      bytes_accessed=bytes_accessed),
    )(xt, w1, b1t, w2s, b2t)
    return qt[:, :B].T


# speedup vs baseline: 1.2363x; 1.2363x over previous
import jax
import jax.numpy as jnp
from jax import lax
from jax.experimental import pallas as pl
from jax.experimental.pallas import tpu as pltpu

_ACTIONS = 8
_BLOCK_N = 65536


def _mlp_t_kernel(xt_ref, w1_ref, b1t_ref, w2s_ref, b2t_ref, o_ref):
    ht = lax.dot_general(w1_ref[...], xt_ref[...],
                         (((0,), (0,)), ((), ())),
                         preferred_element_type=jnp.float32)
    ht = jnp.tanh(ht + b1t_ref[...])
    qt = lax.dot_general(w2s_ref[...], ht,
                         (((0,), (0,)), ((), ())),
                         preferred_element_type=jnp.float32)
    o_ref[...] = (qt + b2t_ref[...]).astype(o_ref.dtype)


def kernel(x, w1, b1, w2p, b2p):
    B, F = x.shape
    H = w1.shape[1]

    xt = x.T
    b1t = b1.T
    w2s = w2p[:, :_ACTIONS]
    b2t = b2p[:, :_ACTIONS].T

    bn = min(_BLOCK_N, B)
    nb = pl.cdiv(B, bn)
    bp = nb * bn
    if bp != B:
        xt = jnp.pad(xt, ((0, 0), (0, bp - B)))

    flops = 2 * bp * (F * H + H * _ACTIONS)
    bytes_accessed = 4 * (bp * F + F * H + H + H * _ACTIONS + _ACTIONS
                          + bp * _ACTIONS)
    qt = pl.pallas_call(
        _mlp_t_kernel,
        out_shape=jax.ShapeDtypeStruct((_ACTIONS, bp), jnp.float32),
        grid=(nb,),
        in_specs=[
            pl.BlockSpec((F, bn), lambda i: (0, i)),
            pl.BlockSpec((F, H), lambda i: (0, 0)),
            pl.BlockSpec((H, 1), lambda i: (0, 0)),
            pl.BlockSpec((H, _ACTIONS), lambda i: (0, 0)),
            pl.BlockSpec((_ACTIONS, 1), lambda i: (0, 0)),
        ],
        out_specs=pl.BlockSpec((_ACTIONS, bn), lambda i: (0, i)),
        compiler_params=pltpu.CompilerParams(
            dimension_semantics=("parallel",)),
        cost_estimate=pl.CostEstimate(flops=flops,
                                      transcendentals=bp * H,
                                      bytes_accessed=bytes_accessed),
    )(xt, w1, b1t, w2s, b2t)
    return qt[:, :B].T


# speedup vs baseline: 1.4096x; 1.1402x over previous
import jax
import jax.numpy as jnp
from jax import lax
from jax.experimental import pallas as pl
from jax.experimental.pallas import tpu as pltpu

_ACTIONS = 8
_BLOCK_N = 65536


def _mlp_t_kernel(xt_ref, w1_ref, b1_ref, w2s_ref, b2_ref, o_ref):
    ht = lax.dot_general(w1_ref[...], xt_ref[...],
                         (((0,), (0,)), ((), ())),
                         preferred_element_type=jnp.float32)
    ht = jnp.tanh(ht + b1_ref[...].T)
    qt = lax.dot_general(w2s_ref[...], ht,
                         (((0,), (0,)), ((), ())),
                         preferred_element_type=jnp.float32)
    o_ref[...] = (qt + b2_ref[...][:, :_ACTIONS].T).astype(o_ref.dtype)


def kernel(x, w1, b1, w2p, b2p):
    B, F = x.shape
    H = w1.shape[1]

    xt = x.T
    w2s = w2p[:, :_ACTIONS]

    bn = min(_BLOCK_N, B)
    nb = pl.cdiv(B, bn)
    bp = nb * bn
    if bp != B:
        xt = jnp.pad(xt, ((0, 0), (0, bp - B)))

    flops = 2 * bp * (F * H + H * _ACTIONS)
    bytes_accessed = 4 * (bp * F + F * H + H + H * _ACTIONS + _ACTIONS
                          + bp * _ACTIONS)
    qt = pl.pallas_call(
        _mlp_t_kernel,
        out_shape=jax.ShapeDtypeStruct((_ACTIONS, bp), jnp.float32),
        grid=(nb,),
        in_specs=[
            pl.BlockSpec((F, bn), lambda i: (0, i)),
            pl.BlockSpec((F, H), lambda i: (0, 0)),
            pl.BlockSpec((1, H), lambda i: (0, 0)),
            pl.BlockSpec((H, _ACTIONS), lambda i: (0, 0)),
            pl.BlockSpec((1, w2p.shape[1]), lambda i: (0, 0)),
        ],
        out_specs=pl.BlockSpec((_ACTIONS, bn), lambda i: (0, i)),
        compiler_params=pltpu.CompilerParams(
            dimension_semantics=("parallel",)),
        cost_estimate=pl.CostEstimate(flops=flops,
                                      transcendentals=bp * H,
                                      bytes_accessed=bytes_accessed),
    )(xt, w1, b1, w2s, b2p)
    return qt[:, :B].T
